# Initial kernel scaffold; baseline (speedup 1.0000x reference)
import jax
import jax.numpy as jnp
from jax.experimental import pallas as pl


def kernel(x, table, W1, b1, W2, b2):
    raise NotImplementedError("write your pallas kernel here")


# speedup vs baseline: 18.2170x; 18.2170x over previous
import functools

import jax
import jax.numpy as jnp
from jax import lax
from jax.experimental import pallas as pl
from jax.experimental.pallas import tpu as pltpu
from jax.experimental.pallas import tpu_sc as plsc

_VOCAB = 2048
_DIM = 2048
_SEQ = 50
_NCLASS = 20
_CPAD = 32
_ROWS = _VOCAB + 8
_NC = 2
_NS = 16
_NW = _NC * _NS
_L = 16


def _project_body(table_ref, w1_ref, w2_ref, b1_ref, b2_ref, out_ref):
    t = table_ref[...]
    h = lax.dot_general(t, w1_ref[...], (((1,), (1,)), ((), ())),
                        preferred_element_type=jnp.float32)
    proj = lax.dot_general(h, w2_ref[...], (((1,), (1,)), ((), ())),
                           preferred_element_type=jnp.float32)
    out_ref[0:_VOCAB, :] = proj * (1.0 / _SEQ)
    brow = lax.dot_general(b1_ref[...], w2_ref[...], (((1,), (1,)), ((), ())),
                           preferred_element_type=jnp.float32) + b2_ref[...]
    out_ref[_VOCAB:_ROWS, :] = jnp.broadcast_to(brow, (_ROWS - _VOCAB, _CPAD))


def _project_table(table, w1, w2p, b1, b2p):
    return pl.pallas_call(
        _project_body,
        out_shape=jax.ShapeDtypeStruct((_ROWS, _CPAD), jnp.float32),
    )(table, w1, w2p, b1.reshape(1, -1), b2p)


def _sc_pool(tblp, idx3, sp, bpw):
    mesh = plsc.VectorSubcoreMesh(core_axis_name="c", subcore_axis_name="s")

    @functools.partial(
        pl.kernel,
        mesh=mesh,
        out_type=jax.ShapeDtypeStruct((_NW, _CPAD, bpw), jnp.float32),
        compiler_params=pltpu.CompilerParams(needs_layout_passes=False),
        scratch_types=[
            pltpu.VMEM((_ROWS * _CPAD,), jnp.float32),
            pltpu.VMEM((sp, bpw), jnp.int32),
            pltpu.VMEM((_CPAD, bpw), jnp.float32),
        ],
    )
    def pool(tbl_hbm, idx_hbm, out_hbm, tbl_v, idx_v, acc_v):
        wid = lax.axis_index("s") * _NC + lax.axis_index("c")
        pltpu.sync_copy(tbl_hbm, tbl_v)
        pltpu.sync_copy(idx_hbm.at[wid], idx_v)
        zeros = jnp.zeros((_L,), jnp.float32)
        for c in range(_CPAD):
            for h in range(bpw // _L):
                acc_v[c, pl.ds(h * _L, _L)] = zeros
        for g in range(bpw // _L):
            def body(s, carry, _g=g):
                rows = idx_v[s, pl.ds(_g * _L, _L)]
                base = rows * _CPAD
                for c in range(_CPAD):
                    v = plsc.load_gather(tbl_v, [base + c])
                    plsc.addupdate(acc_v.at[c, pl.ds(_g * _L, _L)], v)
                return carry
            lax.fori_loop(0, sp, body, 0)
        pltpu.sync_copy(acc_v, out_hbm.at[wid])

    return pool(tblp, idx3)


def kernel(x, table, W1, b1, W2, b2):
    b, s = x.shape
    bpw = b // _NW
    sp = s + 1
    w2p = jnp.zeros((_CPAD, W2.shape[1]), jnp.float32).at[: W2.shape[0]].set(W2)
    b2p = jnp.zeros((1, _CPAD), jnp.float32).at[0, : b2.shape[0]].set(b2)
    tblp = _project_table(table, W1, w2p, b1, b2p)

    xa = jnp.concatenate([x, jnp.full((b, 1), _VOCAB, jnp.int32)], axis=1)
    idx3 = xa.T.reshape(sp, _NW, bpw).transpose(1, 0, 2)

    outw = _sc_pool(tblp.reshape(-1), idx3, sp, bpw)
    out = outw.transpose(0, 2, 1).reshape(b, _CPAD)
    return out[:, :_NCLASS]


# speedup vs baseline: 75.7660x; 4.1591x over previous
import functools

import jax
import jax.numpy as jnp
from jax import lax
from jax.experimental import pallas as pl
from jax.experimental.pallas import tpu as pltpu
from jax.experimental.pallas import tpu_sc as plsc

_VOCAB = 2048
_DIM = 2048
_SEQ = 50
_NCLASS = 20
_CW = 20
_STRIDE = 21
_ROWS = _VOCAB + 8
_NC = 2
_NS = 16
_NW = _NC * _NS
_L = 16


def _project_body(table_ref, w1_ref, w2_ref, b1_ref, b2_ref, out_ref):
    t = table_ref[...]
    h = lax.dot_general(t, w1_ref[...], (((1,), (1,)), ((), ())),
                        preferred_element_type=jnp.float32)
    proj = lax.dot_general(h, w2_ref[...], (((1,), (1,)), ((), ())),
                           preferred_element_type=jnp.float32)
    out_ref[0:_VOCAB, :] = proj * (1.0 / _SEQ)
    brow = lax.dot_general(b1_ref[...], w2_ref[...], (((1,), (1,)), ((), ())),
                           preferred_element_type=jnp.float32) + b2_ref[...]
    out_ref[_VOCAB:_ROWS, :] = jnp.broadcast_to(brow, (_ROWS - _VOCAB, _STRIDE))


def _project_table(table, w1, w2p, b1, b2p):
    return pl.pallas_call(
        _project_body,
        out_shape=jax.ShapeDtypeStruct((_ROWS, _STRIDE), jnp.float32),
    )(table, w1, w2p, b1.reshape(1, -1), b2p)


def _sc_pool(tblp_flat, idx3, sp, bpw):
    mesh = plsc.VectorSubcoreMesh(core_axis_name="c", subcore_axis_name="s")

    @functools.partial(
        pl.kernel,
        mesh=mesh,
        out_type=jax.ShapeDtypeStruct((_NW, _CW, bpw), jnp.float32),
        compiler_params=pltpu.CompilerParams(needs_layout_passes=False),
        scratch_types=[
            pltpu.VMEM((_ROWS * _STRIDE,), jnp.float32),
            pltpu.VMEM((sp, bpw), jnp.int32),
            pltpu.VMEM((_CW, bpw), jnp.float32),
        ],
    )
    def pool(tbl_hbm, idx_hbm, out_hbm, tbl_v, idx_v, acc_v):
        wid = lax.axis_index("s") * _NC + lax.axis_index("c")
        pltpu.sync_copy(tbl_hbm, tbl_v)
        pltpu.sync_copy(idx_hbm.at[wid], idx_v)
        zero = jnp.zeros((_L,), jnp.float32)
        for g in range(bpw // _L):
            def body(s, carry, _g=g):
                rows = idx_v[s, pl.ds(_g * _L, _L)]
                base = rows * _STRIDE
                return tuple(carry[c] + plsc.load_gather(tbl_v, [base + c])
                             for c in range(_CW))
            acc = lax.fori_loop(0, sp, body, (zero,) * _CW)
            for c in range(_CW):
                acc_v[c, pl.ds(g * _L, _L)] = acc[c]
        pltpu.sync_copy(acc_v, out_hbm.at[wid])

    return pool(tblp_flat, idx3)


def kernel(x, table, W1, b1, W2, b2):
    b, s = x.shape
    bpw = b // _NW
    sp = s + 1
    w2p = jnp.zeros((_STRIDE, W2.shape[1]), jnp.float32).at[: W2.shape[0]].set(W2)
    b2p = jnp.zeros((1, _STRIDE), jnp.float32).at[0, : b2.shape[0]].set(b2)
    tblp = _project_table(table, W1, w2p, b1, b2p)

    xa = jnp.concatenate([x, jnp.full((b, 1), _VOCAB, jnp.int32)], axis=1)
    idx3 = xa.T.reshape(sp, _NW, bpw).transpose(1, 0, 2)

    outw = _sc_pool(tblp.reshape(-1), idx3, sp, bpw)
    return outw.transpose(0, 2, 1).reshape(b, _CW)
